# baseline (speedup 1.0000x reference)
import functools

import jax
import jax.numpy as jnp
from jax import lax
from jax.experimental import pallas as pl
from jax.experimental.pallas import tpu as pltpu

B, S, H, E, TOP_K = 4, 8192, 2048, 64, 8
CHUNK = 1024
J = S // CHUNK


def _gate_body(x_ref, fc1w_hbm, fc1b_ref, fc2t_hbm, fc2b_ref,
               idx_ref, w_ref, acc_ref, x1_ref, fc1w_v, fc2t_v, wsem):
    b = pl.program_id(0)
    j = pl.program_id(1)

    @pl.when((b == 0) & (j == 0))
    def _init():
        acc_ref[...] = jnp.zeros_like(acc_ref)

    @pl.when((b == 0) & (j == 1))
    def _prefetch_fc2():
        pltpu.make_async_copy(fc2t_hbm, fc2t_v, wsem).start()

    @pl.when((b == 0) & (j >= 1) & (j <= 4))
    def _prefetch_fc1_piece():
        o = (j - 1) * (H // 4)
        pltpu.make_async_copy(fc1w_hbm.at[pl.ds(o, H // 4), :],
                              fc1w_v.at[pl.ds(o, H // 4), :], wsem).start()

    acc_ref[pl.ds(b, 1), :] += jnp.sum(x_ref[0], axis=0, keepdims=True)

    @pl.when((b == B - 1) & (j == J - 2))
    def _fc1_early():
        pltpu.make_async_copy(fc1w_hbm, fc1w_v, wsem).wait()
        x1_ref[...] = jnp.dot(acc_ref[...] * (1.0 / S), fc1w_v[...],
                              preferred_element_type=jnp.float32
                              ) + fc1b_ref[...]

    @pl.when((b == B - 1) & (j == J - 1))
    def _final():
        pltpu.make_async_copy(fc2t_hbm, fc2t_v, wsem).wait()
        last = jnp.sum(x_ref[0], axis=0, keepdims=True) * (1.0 / S)
        delta = jnp.dot(last, fc1w_v[...],
                        preferred_element_type=jnp.float32)
        rowmask = (lax.broadcasted_iota(jnp.int32, (B, 1), 0) == B - 1
                   ).astype(jnp.float32)
        x = x1_ref[...] + rowmask * delta
        x = 0.5 * x * (1.0 + lax.erf(x * 0.7071067811865476))
        logits = lax.dot_general(
            x, fc2t_v[...], (((1,), (1,)), ((), ())),
            preferred_element_type=jnp.float32) + fc2b_ref[...]
        m = jnp.max(logits, axis=1, keepdims=True)
        e = jnp.exp(logits - m)
        probs = e / jnp.sum(e, axis=1, keepdims=True)

        iota = lax.broadcasted_iota(jnp.int32, (B, E), 1)
        neg = jnp.float32(-jnp.inf)
        p = probs
        vals, idxs = [], []
        for _ in range(TOP_K):
            mv = jnp.max(p, axis=1, keepdims=True)
            first = jnp.min(jnp.where(p >= mv, iota, E), axis=1,
                            keepdims=True)
            vals.append(mv)
            idxs.append(first)
            p = jnp.where(iota == first, neg, p)
        topv = jnp.concatenate(vals, axis=1)
        topi = jnp.concatenate(idxs, axis=1)
        ew = jnp.exp(topv - topv[:, :1])
        w = ew / jnp.sum(ew, axis=1, keepdims=True)
        idx_ref[...] = topi
        w_ref[...] = w


def _gate(hidden_states, fc1_w, fc1_b, fc2_t, fc2_b):
    return pl.pallas_call(
        _gate_body,
        grid=(B, J),
        in_specs=[
            pl.BlockSpec((1, CHUNK, H), lambda b, j: (b, j, 0)),
            pl.BlockSpec(memory_space=pl.ANY),
            pl.BlockSpec((1, H), lambda b, j: (0, 0)),
            pl.BlockSpec(memory_space=pl.ANY),
            pl.BlockSpec((1, E), lambda b, j: (0, 0)),
        ],
        out_specs=[
            pl.BlockSpec((B, TOP_K), lambda b, j: (0, 0)),
            pl.BlockSpec((B, TOP_K), lambda b, j: (0, 0)),
        ],
        out_shape=[
            jax.ShapeDtypeStruct((B, TOP_K), jnp.int32),
            jax.ShapeDtypeStruct((B, TOP_K), jnp.float32),
        ],
        scratch_shapes=[
            pltpu.VMEM((B, H), jnp.float32),
            pltpu.VMEM((B, H), jnp.float32),
            pltpu.VMEM((H, H), jnp.float32),
            pltpu.VMEM((E, H), jnp.float32),
            pltpu.SemaphoreType.DMA,
        ],
        compiler_params=pltpu.CompilerParams(
            dimension_semantics=("arbitrary", "arbitrary"),
        ),
    )(hidden_states, fc1_w, fc1_b.reshape(1, H), fc2_t, fc2_b.reshape(1, E))


def kernel(hidden_states, fc1_w, fc1_b, fc2_w, fc2_b):
    topk_idx, topk_weight = _gate(hidden_states, fc1_w, fc1_b,
                                  fc2_w.T, fc2_b)
    return (topk_idx, topk_weight, jnp.float32(0.0))


# speedup vs baseline: 1.0178x; 1.0178x over previous
import functools

import jax
import jax.numpy as jnp
from jax import lax
from jax.experimental import pallas as pl
from jax.experimental.pallas import tpu as pltpu

B, S, H, E, TOP_K = 4, 8192, 2048, 64, 8
CHUNK = 1024
J = S // CHUNK


def _gate_body(x_ref, fc1w_hbm, fc1b_ref, fc2t_hbm, fc2b_ref,
               idx_ref, w_ref, acc_ref, fc1w_v, fc2t_v, wsem):
    b = pl.program_id(0)
    j = pl.program_id(1)

    @pl.when((b == 0) & (j == 0))
    def _init():
        acc_ref[...] = jnp.zeros_like(acc_ref)

    @pl.when((b == 0) & (j == 1))
    def _prefetch_fc2():
        pltpu.make_async_copy(fc2t_hbm, fc2t_v, wsem).start()

    @pl.when((b == 0) & (j >= 1) & (j <= 4))
    def _prefetch_fc1_piece():
        o = (j - 1) * (H // 4)
        pltpu.make_async_copy(fc1w_hbm.at[pl.ds(o, H // 4), :],
                              fc1w_v.at[pl.ds(o, H // 4), :], wsem).start()

    acc_ref[pl.ds(b, 1), :] += jnp.sum(x_ref[0], axis=0, keepdims=True)

    @pl.when((b == B - 1) & (j == J - 1))
    def _final():
        pltpu.make_async_copy(fc1w_hbm, fc1w_v, wsem).wait()
        pltpu.make_async_copy(fc2t_hbm, fc2t_v, wsem).wait()
        seq = acc_ref[...] * (1.0 / S)
        x = jnp.dot(seq, fc1w_v[...],
                    preferred_element_type=jnp.float32) + fc1b_ref[...]
        x = 0.5 * x * (1.0 + lax.erf(x * 0.7071067811865476))
        logits = lax.dot_general(
            x, fc2t_v[...], (((1,), (1,)), ((), ())),
            preferred_element_type=jnp.float32) + fc2b_ref[...]
        m = jnp.max(logits, axis=1, keepdims=True)
        e = jnp.exp(logits - m)
        probs = e / jnp.sum(e, axis=1, keepdims=True)

        iota = lax.broadcasted_iota(jnp.int32, (B, E), 1)
        pa = probs[:, :, None]
        pb = probs[:, None, :]
        ia = iota[:, :, None]
        ib = iota[:, None, :]
        ahead = (pb > pa) | ((pb == pa) & (ib < ia))
        rank = jnp.sum(ahead.astype(jnp.float32), axis=2)
        vals, idxs = [], []
        for k in range(TOP_K):
            mk = (rank == k).astype(jnp.float32)
            vals.append(jnp.sum(probs * mk, axis=1, keepdims=True))
            idxs.append(jnp.sum(iota * mk.astype(jnp.int32), axis=1,
                                keepdims=True))
        topv = jnp.concatenate(vals, axis=1)
        topi = jnp.concatenate(idxs, axis=1)
        ew = jnp.exp(topv - topv[:, :1])
        w = ew / jnp.sum(ew, axis=1, keepdims=True)
        idx_ref[...] = topi
        w_ref[...] = w


def _gate(hidden_states, fc1_w, fc1_b, fc2_t, fc2_b):
    return pl.pallas_call(
        _gate_body,
        grid=(B, J),
        in_specs=[
            pl.BlockSpec((1, CHUNK, H), lambda b, j: (b, j, 0)),
            pl.BlockSpec(memory_space=pl.ANY),
            pl.BlockSpec((1, H), lambda b, j: (0, 0)),
            pl.BlockSpec(memory_space=pl.ANY),
            pl.BlockSpec((1, E), lambda b, j: (0, 0)),
        ],
        out_specs=[
            pl.BlockSpec((B, TOP_K), lambda b, j: (0, 0)),
            pl.BlockSpec((B, TOP_K), lambda b, j: (0, 0)),
        ],
        out_shape=[
            jax.ShapeDtypeStruct((B, TOP_K), jnp.int32),
            jax.ShapeDtypeStruct((B, TOP_K), jnp.float32),
        ],
        scratch_shapes=[
            pltpu.VMEM((B, H), jnp.float32),
            pltpu.VMEM((H, H), jnp.float32),
            pltpu.VMEM((E, H), jnp.float32),
            pltpu.SemaphoreType.DMA,
        ],
        compiler_params=pltpu.CompilerParams(
            dimension_semantics=("arbitrary", "arbitrary"),
        ),
    )(hidden_states, fc1_w, fc1_b.reshape(1, H), fc2_t, fc2_b.reshape(1, E))


def kernel(hidden_states, fc1_w, fc1_b, fc2_w, fc2_b):
    topk_idx, topk_weight = _gate(hidden_states, fc1_w, fc1_b,
                                  fc2_w.T, fc2_b)
    return (topk_idx, topk_weight, jnp.float32(0.0))
